# Initial kernel scaffold; baseline (speedup 1.0000x reference)
import jax
import jax.numpy as jnp
from jax.experimental import pallas as pl


def kernel(x, y):
    raise NotImplementedError("write your pallas kernel here")


# speedup vs baseline: 98.2065x; 98.2065x over previous
import functools
import math

import jax
import jax.numpy as jnp
from jax.experimental import pallas as pl
from jax.experimental.pallas import tpu as pltpu

_LOG_THRESH = math.log(1000.0)
_ONE_MINUS_INV_E = 1.0 - math.exp(-1.0)


def _loss_kernel(n_classes, y_ref, x_ref, out_ref,
                 m1_ref, m2_ref, idx_ref, s_ref, g_ref):
    j = pl.program_id(0)
    nblk = pl.num_programs(0)
    bsz, bc = x_ref.shape
    base = j * bc
    neg_inf = jnp.float32(-jnp.inf)

    xb = x_ref[...]
    col = jax.lax.broadcasted_iota(jnp.int32, (1, bc), 1) + base
    xb = jnp.where(col < n_classes, xb, neg_inf)

    yv = y_ref[...]
    eq = col == yv
    g_part = jnp.sum(jnp.where(eq, xb, 0.0), axis=1, keepdims=True)

    bm1 = jnp.max(xb, axis=1, keepdims=True)
    big = jnp.int32(2 ** 30)
    fidx = jnp.min(jnp.where(xb == bm1, col, big), axis=1, keepdims=True)
    bm2 = jnp.max(jnp.where(col == fidx, neg_inf, xb), axis=1, keepdims=True)
    bs = jnp.sum(jnp.exp(xb - bm1), axis=1, keepdims=True)

    @pl.when(j == 0)
    def _init():
        m1_ref[...] = bm1
        m2_ref[...] = bm2
        idx_ref[...] = fidx
        s_ref[...] = bs
        g_ref[...] = g_part

    @pl.when(j > 0)
    def _acc():
        r1 = m1_ref[...]
        r2 = m2_ref[...]
        n1 = jnp.maximum(r1, bm1)
        n2 = jnp.maximum(jnp.minimum(r1, bm1), jnp.maximum(r2, bm2))
        s_ref[...] = s_ref[...] * jnp.exp(r1 - n1) + bs * jnp.exp(bm1 - n1)
        idx_ref[...] = jnp.where(bm1 > r1, fidx, idx_ref[...])
        m1_ref[...] = n1
        m2_ref[...] = n2
        g_ref[...] = g_ref[...] + g_part

    @pl.when(j == nblk - 1)
    def _finish():
        m1 = m1_ref[...]
        m2 = m2_ref[...]
        idx = idx_ref[...]
        s = s_ref[...]
        g = g_ref[...]

        hard = ((m1 - m2) >= jnp.float32(_LOG_THRESH)).astype(jnp.float32)

        lse = m1 + 1.0 + jnp.log(s - jnp.exp(g - m1) * jnp.float32(_ONE_MINUS_INV_E))
        smooth_loss = lse - g

        mex = jnp.where(idx == y_ref[...], m2, m1)
        hard_loss = jnp.maximum(mex + 1.0, g) - g

        n_hard = jnp.sum(hard)
        n_smooth = jnp.float32(bsz) - n_hard
        hard_sum = jnp.sum(hard_loss * hard)
        smooth_sum = jnp.sum(smooth_loss * (1.0 - hard))

        loss = (jnp.where(n_smooth > 0, smooth_sum / jnp.maximum(n_smooth, 1.0), 0.0)
                + jnp.where(n_hard > 0, hard_sum / jnp.maximum(n_hard, 1.0), 0.0))
        out_ref[0, 0] = loss


def kernel(x, y):
    b, n = x.shape
    bc = 2048
    nblk = pl.cdiv(n, bc)
    y2 = y.reshape(b, 1).astype(jnp.int32)
    out = pl.pallas_call(
        functools.partial(_loss_kernel, n),
        grid=(nblk,),
        in_specs=[
            pl.BlockSpec((b, 1), lambda j: (0, 0)),
            pl.BlockSpec((b, bc), lambda j: (0, j)),
        ],
        out_specs=pl.BlockSpec(memory_space=pltpu.SMEM),
        out_shape=jax.ShapeDtypeStruct((1, 1), jnp.float32),
        scratch_shapes=[
            pltpu.VMEM((b, 1), jnp.float32),
            pltpu.VMEM((b, 1), jnp.float32),
            pltpu.VMEM((b, 1), jnp.int32),
            pltpu.VMEM((b, 1), jnp.float32),
            pltpu.VMEM((b, 1), jnp.float32),
        ],
        compiler_params=pltpu.CompilerParams(
            dimension_semantics=("arbitrary",),
        ),
    )(y2, x)
    return out[0, 0]
